# baseline (speedup 1.0000x reference)
import functools

import jax
import jax.numpy as jnp
from jax import lax
from jax.experimental import pallas as pl
from jax.experimental.pallas import tpu as pltpu
from jax.experimental.pallas import tpu_sc as plsc

N = 10000
NP = 10240
D = 256
E = 160000
E_PAD = 163840
PAD_DST = 16384
BROWS = 320
NBUK = 32
CAP = 6144
DUMP = 4096
TOT = NBUK * CAP + DUMP
RB = 512
SUB = 8
G = 64
MAGIC = 0x5F3759DF

_MESH = plsc.VectorSubcoreMesh(core_axis_name="c", subcore_axis_name="s")


def _rank_body(src_ref, dst_ref, pos_ref, packed_ref, tot_ref, carry_ref):
    step = pl.program_id(0)

    @pl.when(step == 0)
    def _():
        carry_ref[...] = jnp.zeros((1, 128), jnp.float32)

    col = lax.broadcasted_iota(jnp.int32, (RB, 128), 1)
    rr = lax.broadcasted_iota(jnp.int32, (RB, RB), 0)
    cc = lax.broadcasted_iota(jnp.int32, (RB, RB), 1)
    ltri = (rr > cc).astype(jnp.float32)
    ones = jnp.ones((128, 128), jnp.float32)
    for sb in range(SUB):
        d = dst_ref[pl.ds(sb * RB, RB), :]
        s = src_ref[pl.ds(sb * RB, RB), :]
        b = jnp.minimum((d * 6554) >> 21, NBUK)
        l = d - b * BROWS
        ind = (b == col).astype(jnp.float32)
        rank = jnp.dot(ltri, ind, preferred_element_type=jnp.float32)
        carry = carry_ref[...]
        rel = jnp.sum(ind * (rank + carry), axis=1, keepdims=True)
        carry_ref[...] = carry + jnp.sum(ind, axis=0, keepdims=True)
        pos_ref[pl.ds(sb * RB, RB), :] = rel.astype(jnp.int32) + b * CAP
        packed_ref[pl.ds(sb * RB, RB), :] = s * 512 + l
    tot_ref[...] = carry_ref[...]


def _tc_rank(src2, dst2):
    nstep = E_PAD // (RB * SUB)
    be = pl.BlockSpec((RB * SUB, 1), lambda i: (i, 0))
    bt = pl.BlockSpec((1, 128), lambda i: (0, 0))
    return pl.pallas_call(
        _rank_body,
        grid=(nstep,),
        in_specs=[be, be],
        out_specs=[be, be, bt],
        out_shape=[
            jax.ShapeDtypeStruct((E_PAD, 1), jnp.int32),
            jax.ShapeDtypeStruct((E_PAD, 1), jnp.int32),
            jax.ShapeDtypeStruct((1, 128), jnp.float32),
        ],
        scratch_shapes=[pltpu.VMEM((1, 128), jnp.float32)],
    )(src2, dst2)


@functools.partial(
    pl.kernel,
    out_type=jax.ShapeDtypeStruct((TOT,), jnp.int32),
    mesh=_MESH,
    scratch_types=[
        pltpu.VMEM((128,), jnp.int32),
        pltpu.VMEM((128,), jnp.int32),
        pltpu.VMEM((128,), jnp.int32),
        pltpu.VMEM((128,), jnp.int32),
        pltpu.VMEM((64,), jnp.int32),
        pltpu.VMEM((64,), jnp.int32),
        pltpu.VMEM((16,), jnp.int32),
        pltpu.SemaphoreType.DMA,
        pltpu.SemaphoreType.DMA,
    ],
)
def _sc_scatter(packed_hbm, pos_hbm, cnts_hbm, recs_hbm,
                pos0, pos1, val0, val1, fidx_v, fval_v, cnt_v, sem0, sem1):
    c = lax.axis_index("c")
    s = lax.axis_index("s")
    wid = c * 16 + s
    base = wid * (E_PAD // 32)
    poss = (pos0, pos1)
    vals = (val0, val1)
    sems = (sem0, sem1)

    def body(j, carry):
        for b in range(2):
            off = base + (j * 2 + b) * 128
            pltpu.sync_copy(pos_hbm.at[pl.ds(off, 128)], poss[b])
            pltpu.sync_copy(packed_hbm.at[pl.ds(off, 128)], vals[b])
            pltpu.sync_copy(vals[b], recs_hbm.at[poss[b]])
        return carry

    nchunk = E_PAD // 32 // 128
    lax.fori_loop(0, nchunk // 2, body, 0)

    pltpu.sync_copy(cnts_hbm.at[pl.ds(wid * 16, 16)], cnt_v)
    cnt = cnt_v[...][0]
    iota16 = lax.iota(jnp.int32, 16)
    dummy = jnp.full((16,), BROWS, jnp.int32)
    for half in range(2):
        for t in range(4):
            fidx_v[pl.ds(t * 16, 16)] = (wid * CAP + cnt + (half * 4 + t) * 16
                                         + iota16)
            fval_v[pl.ds(t * 16, 16)] = dummy
        pltpu.sync_copy(fval_v, recs_hbm.at[fidx_v])


@functools.partial(
    pl.kernel,
    out_type=jax.ShapeDtypeStruct((NP, 16), jnp.float32),
    mesh=_MESH,
    scratch_types=[
        pltpu.VMEM((CAP,), jnp.int32),
        pltpu.VMEM((16,), jnp.int32),
        pltpu.VMEM((336, 16), jnp.float32),
        pltpu.VMEM((BROWS, 16), jnp.float32),
    ],
)
def _sc_degdinv(recs_hbm, cnts_hbm, dinv_hbm, rec_v, cnt_v, deg_v, dv_v):
    c = lax.axis_index("c")
    s = lax.axis_index("s")
    wid = c * 16 + s
    pltpu.sync_copy(recs_hbm.at[pl.ds(wid * CAP, CAP)], rec_v)
    pltpu.sync_copy(cnts_hbm.at[pl.ds(wid * 16, 16)], cnt_v)
    cntp = cnt_v[...][0]
    one16 = jnp.ones((16,), jnp.float32)

    def init_body(k, carry):
        for t in range(4):
            deg_v[k * 4 + t, :] = one16
        return carry

    lax.fori_loop(0, 84, init_body, 0)

    def edge_body(j, carry):
        p16 = rec_v[pl.ds(j * 16, 16)]
        l16 = jnp.minimum(p16 & 511, BROWS)
        for e in range(16):
            le = l16[e]
            deg_v[le, :] = deg_v[le, :] + 1.0
        return carry

    lax.fori_loop(0, cntp // 16, edge_body, 0)

    def dinv_body(k, carry):
        for t in range(4):
            r = k * 4 + t
            dd = deg_v[r, :]
            ib = lax.bitcast_convert_type(dd, jnp.int32)
            ib = jnp.int32(MAGIC) - (ib >> 1)
            x = lax.bitcast_convert_type(ib, jnp.float32)
            for _ in range(3):
                x = x * (1.5 - 0.5 * dd * x * x)
            dv_v[r, :] = x
        return carry

    lax.fori_loop(0, BROWS // 4, dinv_body, 0)
    pltpu.sync_copy(dv_v, dinv_hbm.at[pl.ds(wid * BROWS, BROWS)])


@functools.partial(
    pl.kernel,
    out_type=jax.ShapeDtypeStruct((NP, D), jnp.float32),
    mesh=_MESH,
    scratch_types=[
        pltpu.VMEM((CAP,), jnp.int32),
        pltpu.VMEM((16,), jnp.int32),
        pltpu.VMEM((2 * G,), jnp.int32),
        pltpu.VMEM((G, D), jnp.float32),
        pltpu.VMEM((G, D), jnp.float32),
        pltpu.VMEM((328, D), jnp.float32),
        pltpu.SemaphoreType.DMA,
        pltpu.SemaphoreType.DMA,
    ],
)
def _sc_prop(y_hbm, recs_hbm, cnts_hbm, out_hbm,
             rec_v, cnt_v, gidx_v, stage0, stage1, acc_v, sem0, sem1):
    c = lax.axis_index("c")
    s = lax.axis_index("s")
    wid = c * 16 + s
    base_row = wid * BROWS
    pltpu.sync_copy(recs_hbm.at[pl.ds(wid * CAP, CAP)], rec_v)
    pltpu.sync_copy(cnts_hbm.at[pl.ds(wid * 16, 16)], cnt_v)
    cntp = cnt_v[...][0]
    ngrp = cntp // G
    gmax = jnp.maximum(ngrp - 1, 0)
    pltpu.sync_copy(y_hbm.at[pl.ds(base_row, BROWS)], acc_v.at[pl.ds(0, BROWS)])

    stages = (stage0, stage1)
    sems = (sem0, sem1)

    def fire(g, slot):
        gg = jnp.minimum(g, gmax) * G
        for t in range(G // 16):
            p16 = rec_v[pl.ds(gg + t * 16, 16)]
            gidx_v[pl.ds(slot * G + t * 16, 16)] = jnp.minimum(p16 >> 9, NP - 1)
        pltpu.async_copy(y_hbm.at[gidx_v.at[pl.ds(slot * G, G)]],
                         stages[slot], sems[slot])

    def wait(slot):
        pltpu.make_async_copy(y_hbm.at[gidx_v.at[pl.ds(slot * G, G)]],
                              stages[slot], sems[slot]).wait()

    def process(g, slot):
        st = stages[slot]
        les = []
        for t in range(G // 16):
            p16 = rec_v[pl.ds(g * G + t * 16, 16)]
            l16 = jnp.minimum(p16 & 511, BROWS)
            for e in range(16):
                les.append(l16[e])
        for e in range(G):
            le = les[e]
            vals = [acc_v[le, pl.ds(k * 16, 16)] for k in range(16)]
            svals = [st[e, pl.ds(k * 16, 16)] for k in range(16)]
            for k in range(16):
                acc_v[le, pl.ds(k * 16, 16)] = vals[k] + svals[k]

    @pl.when(ngrp > 0)
    def _():
        fire(0, 0)
        fire(1, 1)

        def body(j, carry):
            for b in range(2):
                g = j * 2 + b
                wait(b)
                process(g, b)
                fire(g + 2, b)
            return carry

        lax.fori_loop(0, ngrp // 2, body, 0)
        wait(0)
        wait(1)

    pltpu.sync_copy(acc_v.at[pl.ds(0, BROWS)],
                    out_hbm.at[pl.ds(base_row, BROWS)])


_BM = 1024


def _scale3_body(inp_ref, h0_ref, h1_ref, dinv_ref, yx_ref, yh0_ref, yh1_ref):
    dv = dinv_ref[...]
    yx_ref[...] = inp_ref[...] * dv
    yh0_ref[...] = h0_ref[...] * dv
    yh1_ref[...] = h1_ref[...] * dv


def _tc_scale3(inp, h0, h1, dinv2):
    bs = pl.BlockSpec((_BM, D), lambda i: (i, 0))
    bd = pl.BlockSpec((_BM, 1), lambda i: (i, 0))
    return pl.pallas_call(
        _scale3_body,
        grid=(NP // _BM,),
        in_specs=[bs, bs, bs, bd],
        out_specs=[bs, bs, bs],
        out_shape=[jax.ShapeDtypeStruct((NP, D), jnp.float32)] * 3,
    )(inp, h0, h1, dinv2)


def _zr_body(accx_ref, acch_ref, dinv_ref, h_ref, wx_ref, wh_ref, b_ref,
             z_ref, yrh_ref):
    dv = dinv_ref[...]
    px = accx_ref[...] * dv
    ph = acch_ref[...] * dv
    sv = (jnp.dot(px, wx_ref[...], preferred_element_type=jnp.float32)
          + jnp.dot(ph, wh_ref[...], preferred_element_type=jnp.float32)
          + b_ref[0:1, :])
    zr = jax.nn.sigmoid(sv)
    z_ref[...] = zr[:, :D]
    yrh_ref[...] = dv * (zr[:, D:] * h_ref[...])


def _tc_zr(accx, acch, dinv2, h_i, wx, wh, b2):
    bs = pl.BlockSpec((_BM, D), lambda i: (i, 0))
    bd = pl.BlockSpec((_BM, 1), lambda i: (i, 0))
    bw = pl.BlockSpec((D, 2 * D), lambda i: (0, 0))
    bb = pl.BlockSpec((8, 2 * D), lambda i: (0, 0))
    return pl.pallas_call(
        _zr_body,
        grid=(NP // _BM,),
        in_specs=[bs, bs, bd, bs, bw, bw, bb],
        out_specs=[bs, bs],
        out_shape=[jax.ShapeDtypeStruct((NP, D), jnp.float32)] * 2,
    )(accx, acch, dinv2, h_i, wx, wh, b2)


def _ht_body(accx_ref, accrh_ref, dinv_ref, z_ref, h_ref, wx_ref, wh_ref,
             b_ref, hn_ref, y_ref):
    dv = dinv_ref[...]
    px = accx_ref[...] * dv
    prh = accrh_ref[...] * dv
    sv = (jnp.dot(px, wx_ref[...], preferred_element_type=jnp.float32)
          + jnp.dot(prh, wh_ref[...], preferred_element_type=jnp.float32)
          + b_ref[0:1, :])
    ht = jnp.tanh(sv)
    z = z_ref[...]
    hn = z * h_ref[...] + (1.0 - z) * ht
    hn_ref[...] = hn
    y_ref[...] = dv * hn


def _tc_ht(accx, accrh, dinv2, z, h_i, wx, wh, b2):
    bs = pl.BlockSpec((_BM, D), lambda i: (i, 0))
    bd = pl.BlockSpec((_BM, 1), lambda i: (i, 0))
    bw = pl.BlockSpec((D, D), lambda i: (0, 0))
    bb = pl.BlockSpec((8, D), lambda i: (0, 0))
    return pl.pallas_call(
        _ht_body,
        grid=(NP // _BM,),
        in_specs=[bs, bs, bd, bs, bs, bw, bw, bb],
        out_specs=[bs, bs],
        out_shape=[jax.ShapeDtypeStruct((NP, D), jnp.float32)] * 2,
    )(accx, accrh, dinv2, z, h_i, wx, wh, b2)


def kernel(inp, edgidx, h, Wxz, bxz, Whz, bhz, Wxr, bxr, Whr, bhr,
           Wxh, bxh, Whh, bhh):
    src = edgidx[0].astype(jnp.int32)
    dst = edgidx[1].astype(jnp.int32)
    pad = E_PAD - E
    src2 = jnp.concatenate([src, jnp.zeros((pad,), jnp.int32)]).reshape(E_PAD, 1)
    dst2 = jnp.concatenate([dst, jnp.full((pad,), PAD_DST, jnp.int32)]
                           ).reshape(E_PAD, 1)

    pos2, packed2, totals = _tc_rank(src2, dst2)
    counts = totals[0, :NBUK].astype(jnp.int32)
    counts_pad = ((counts + 127) // 128) * 128
    cnts_rep = jnp.repeat(counts, 16)
    cnts_pad_rep = jnp.repeat(counts_pad, 16)

    recs = _sc_scatter(packed2.reshape(-1), pos2.reshape(-1), cnts_rep)
    dinv_rep = _sc_degdinv(recs, cnts_pad_rep)
    dinv2 = dinv_rep[:, :1]

    rowpad = ((0, NP - N), (0, 0))
    yx, yh0, yh1 = _tc_scale3(jnp.pad(inp, rowpad), jnp.pad(h[0], rowpad),
                              jnp.pad(h[1], rowpad), dinv2)

    h_new = []
    yhs = (yh0, yh1)
    y_in = yx
    for i in range(2):
        wx_zr = jnp.concatenate([Wxz[i], Wxr[i]], axis=1)
        wh_zr = jnp.concatenate([Whz[i], Whr[i]], axis=1)
        b_zr = jnp.concatenate([bxz[i] + bhz[i], bxr[i] + bhr[i]])
        b_zr2 = jnp.broadcast_to(b_zr[None, :], (8, 2 * D))
        b_h2 = jnp.broadcast_to((bxh[i] + bhh[i])[None, :], (8, D))
        h_i = jnp.pad(h[i], rowpad)

        accx = _sc_prop(y_in, recs, cnts_pad_rep)
        acch = _sc_prop(yhs[i], recs, cnts_pad_rep)
        z, yrh = _tc_zr(accx, acch, dinv2, h_i, wx_zr, wh_zr, b_zr2)
        accrh = _sc_prop(yrh, recs, cnts_pad_rep)
        hn, y_in = _tc_ht(accx, accrh, dinv2, z, h_i, Wxh[i], Whh[i], b_h2)
        h_new.append(hn[:N])

    h_out = jnp.stack(h_new, axis=0)
    return (h_out, h_out)


# speedup vs baseline: 1.1141x; 1.1141x over previous
import functools

import jax
import jax.numpy as jnp
from jax import lax
from jax.experimental import pallas as pl
from jax.experimental.pallas import tpu as pltpu
from jax.experimental.pallas import tpu_sc as plsc

N = 10000
NP = 10240
D = 256
E = 160000
E_PAD = 163840
PAD_DST = 16384
BROWS = 320
NBUK = 32
CAP = 6144
DUMP = 4096
TOT = NBUK * CAP + DUMP
RB = 512
SUB = 8
G = 32
MAGIC = 0x5F3759DF

_MESH = plsc.VectorSubcoreMesh(core_axis_name="c", subcore_axis_name="s")


def _rank_body(src_ref, dst_ref, pos_ref, packed_ref, tot_ref, carry_ref):
    step = pl.program_id(0)

    @pl.when(step == 0)
    def _():
        carry_ref[...] = jnp.zeros((1, 128), jnp.float32)

    col = lax.broadcasted_iota(jnp.int32, (RB, 128), 1)
    rr = lax.broadcasted_iota(jnp.int32, (RB, RB), 0)
    cc = lax.broadcasted_iota(jnp.int32, (RB, RB), 1)
    ltri = (rr > cc).astype(jnp.float32)
    ones = jnp.ones((128, 128), jnp.float32)
    for sb in range(SUB):
        d = dst_ref[pl.ds(sb * RB, RB), :]
        s = src_ref[pl.ds(sb * RB, RB), :]
        b = jnp.minimum((d * 6554) >> 21, NBUK)
        l = d - b * BROWS
        ind = (b == col).astype(jnp.float32)
        rank = jnp.dot(ltri, ind, preferred_element_type=jnp.float32)
        carry = carry_ref[...]
        rel = jnp.sum(ind * (rank + carry), axis=1, keepdims=True)
        carry_ref[...] = carry + jnp.sum(ind, axis=0, keepdims=True)
        pos_ref[pl.ds(sb * RB, RB), :] = rel.astype(jnp.int32) + b * CAP
        packed_ref[pl.ds(sb * RB, RB), :] = s * 512 + l
    tot_ref[...] = carry_ref[...]


def _tc_rank(src2, dst2):
    nstep = E_PAD // (RB * SUB)
    be = pl.BlockSpec((RB * SUB, 1), lambda i: (i, 0))
    bt = pl.BlockSpec((1, 128), lambda i: (0, 0))
    return pl.pallas_call(
        _rank_body,
        grid=(nstep,),
        in_specs=[be, be],
        out_specs=[be, be, bt],
        out_shape=[
            jax.ShapeDtypeStruct((E_PAD, 1), jnp.int32),
            jax.ShapeDtypeStruct((E_PAD, 1), jnp.int32),
            jax.ShapeDtypeStruct((1, 128), jnp.float32),
        ],
        scratch_shapes=[pltpu.VMEM((1, 128), jnp.float32)],
    )(src2, dst2)


@functools.partial(
    pl.kernel,
    out_type=jax.ShapeDtypeStruct((TOT,), jnp.int32),
    mesh=_MESH,
    scratch_types=[
        pltpu.VMEM((128,), jnp.int32),
        pltpu.VMEM((128,), jnp.int32),
        pltpu.VMEM((128,), jnp.int32),
        pltpu.VMEM((128,), jnp.int32),
        pltpu.VMEM((64,), jnp.int32),
        pltpu.VMEM((64,), jnp.int32),
        pltpu.VMEM((16,), jnp.int32),
        pltpu.SemaphoreType.DMA,
        pltpu.SemaphoreType.DMA,
    ],
)
def _sc_scatter(packed_hbm, pos_hbm, cnts_hbm, recs_hbm,
                pos0, pos1, val0, val1, fidx_v, fval_v, cnt_v, sem0, sem1):
    c = lax.axis_index("c")
    s = lax.axis_index("s")
    wid = c * 16 + s
    base = wid * (E_PAD // 32)
    poss = (pos0, pos1)
    vals = (val0, val1)
    sems = (sem0, sem1)

    def body(j, carry):
        for b in range(2):
            off = base + (j * 2 + b) * 128
            pltpu.sync_copy(pos_hbm.at[pl.ds(off, 128)], poss[b])
            pltpu.sync_copy(packed_hbm.at[pl.ds(off, 128)], vals[b])
            pltpu.sync_copy(vals[b], recs_hbm.at[poss[b]])
        return carry

    nchunk = E_PAD // 32 // 128
    lax.fori_loop(0, nchunk // 2, body, 0)

    pltpu.sync_copy(cnts_hbm.at[pl.ds(wid * 16, 16)], cnt_v)
    cnt = cnt_v[...][0]
    iota16 = lax.iota(jnp.int32, 16)
    dummy = jnp.full((16,), BROWS, jnp.int32)
    for half in range(2):
        for t in range(4):
            fidx_v[pl.ds(t * 16, 16)] = (wid * CAP + cnt + (half * 4 + t) * 16
                                         + iota16)
            fval_v[pl.ds(t * 16, 16)] = dummy
        pltpu.sync_copy(fval_v, recs_hbm.at[fidx_v])


@functools.partial(
    pl.kernel,
    out_type=jax.ShapeDtypeStruct((NP, 16), jnp.float32),
    mesh=_MESH,
    scratch_types=[
        pltpu.VMEM((CAP,), jnp.int32),
        pltpu.VMEM((16,), jnp.int32),
        pltpu.VMEM((336, 16), jnp.float32),
        pltpu.VMEM((BROWS, 16), jnp.float32),
    ],
)
def _sc_degdinv(recs_hbm, cnts_hbm, dinv_hbm, rec_v, cnt_v, deg_v, dv_v):
    c = lax.axis_index("c")
    s = lax.axis_index("s")
    wid = c * 16 + s
    pltpu.sync_copy(recs_hbm.at[pl.ds(wid * CAP, CAP)], rec_v)
    pltpu.sync_copy(cnts_hbm.at[pl.ds(wid * 16, 16)], cnt_v)
    cntp = cnt_v[...][0]
    one16 = jnp.ones((16,), jnp.float32)

    def init_body(k, carry):
        for t in range(4):
            deg_v[k * 4 + t, :] = one16
        return carry

    lax.fori_loop(0, 84, init_body, 0)

    def edge_body(j, carry):
        p16 = rec_v[pl.ds(j * 16, 16)]
        l16 = jnp.minimum(p16 & 511, BROWS)
        for e in range(16):
            le = l16[e]
            deg_v[le, :] = deg_v[le, :] + 1.0
        return carry

    lax.fori_loop(0, cntp // 16, edge_body, 0)

    def dinv_body(k, carry):
        for t in range(4):
            r = k * 4 + t
            dd = deg_v[r, :]
            ib = lax.bitcast_convert_type(dd, jnp.int32)
            ib = jnp.int32(MAGIC) - (ib >> 1)
            x = lax.bitcast_convert_type(ib, jnp.float32)
            for _ in range(3):
                x = x * (1.5 - 0.5 * dd * x * x)
            dv_v[r, :] = x
        return carry

    lax.fori_loop(0, BROWS // 4, dinv_body, 0)
    pltpu.sync_copy(dv_v, dinv_hbm.at[pl.ds(wid * BROWS, BROWS)])


@functools.partial(
    pl.kernel,
    out_type=jax.ShapeDtypeStruct((NP, D), jnp.float32),
    mesh=_MESH,
    scratch_types=[
        pltpu.VMEM((CAP,), jnp.int32),
        pltpu.VMEM((16,), jnp.int32),
        pltpu.VMEM((2 * G,), jnp.int32),
        pltpu.VMEM((G, D), jnp.float32),
        pltpu.VMEM((G, D), jnp.float32),
        pltpu.VMEM((328, D), jnp.float32),
        pltpu.SemaphoreType.DMA,
        pltpu.SemaphoreType.DMA,
    ],
)
def _sc_prop(y_hbm, recs_hbm, cnts_hbm, out_hbm,
             rec_v, cnt_v, gidx_v, stage0, stage1, acc_v, sem0, sem1):
    c = lax.axis_index("c")
    s = lax.axis_index("s")
    wid = c * 16 + s
    base_row = wid * BROWS
    pltpu.sync_copy(recs_hbm.at[pl.ds(wid * CAP, CAP)], rec_v)
    pltpu.sync_copy(cnts_hbm.at[pl.ds(wid * 16, 16)], cnt_v)
    cntp = cnt_v[...][0]
    ngrp = cntp // G
    gmax = jnp.maximum(ngrp - 1, 0)
    pltpu.sync_copy(y_hbm.at[pl.ds(base_row, BROWS)], acc_v.at[pl.ds(0, BROWS)])

    stages = (stage0, stage1)
    sems = (sem0, sem1)

    def fire(g, slot):
        gg = jnp.minimum(g, gmax) * G
        for t in range(G // 16):
            p16 = rec_v[pl.ds(gg + t * 16, 16)]
            gidx_v[pl.ds(slot * G + t * 16, 16)] = jnp.minimum(p16 >> 9, NP - 1)
        pltpu.async_copy(y_hbm.at[gidx_v.at[pl.ds(slot * G, G)]],
                         stages[slot], sems[slot])

    def wait(slot):
        pltpu.make_async_copy(y_hbm.at[gidx_v.at[pl.ds(slot * G, G)]],
                              stages[slot], sems[slot]).wait()

    def process(g, slot):
        st = stages[slot]
        les = []
        for t in range(G // 16):
            p16 = rec_v[pl.ds(g * G + t * 16, 16)]
            l16 = jnp.minimum(p16 & 511, BROWS)
            for e in range(16):
                les.append(l16[e])
        for e in range(G):
            le = les[e]
            vals = [acc_v[le, pl.ds(k * 16, 16)] for k in range(16)]
            svals = [st[e, pl.ds(k * 16, 16)] for k in range(16)]
            for k in range(16):
                acc_v[le, pl.ds(k * 16, 16)] = vals[k] + svals[k]

    @pl.when(ngrp > 0)
    def _():
        fire(0, 0)
        fire(1, 1)

        def body(j, carry):
            for b in range(2):
                g = j * 2 + b
                wait(b)
                process(g, b)
                fire(g + 2, b)
            return carry

        lax.fori_loop(0, ngrp // 2, body, 0)
        wait(0)
        wait(1)

    pltpu.sync_copy(acc_v.at[pl.ds(0, BROWS)],
                    out_hbm.at[pl.ds(base_row, BROWS)])


_BM = 1024


def _scale3_body(inp_ref, h0_ref, h1_ref, dinv_ref, yx_ref, yh0_ref, yh1_ref):
    dv = dinv_ref[...]
    yx_ref[...] = inp_ref[...] * dv
    yh0_ref[...] = h0_ref[...] * dv
    yh1_ref[...] = h1_ref[...] * dv


def _tc_scale3(inp, h0, h1, dinv2):
    bs = pl.BlockSpec((_BM, D), lambda i: (i, 0))
    bd = pl.BlockSpec((_BM, 1), lambda i: (i, 0))
    return pl.pallas_call(
        _scale3_body,
        grid=(NP // _BM,),
        in_specs=[bs, bs, bs, bd],
        out_specs=[bs, bs, bs],
        out_shape=[jax.ShapeDtypeStruct((NP, D), jnp.float32)] * 3,
    )(inp, h0, h1, dinv2)


def _zr_body(accx_ref, acch_ref, dinv_ref, h_ref, wx_ref, wh_ref, b_ref,
             z_ref, yrh_ref):
    dv = dinv_ref[...]
    px = accx_ref[...] * dv
    ph = acch_ref[...] * dv
    sv = (jnp.dot(px, wx_ref[...], preferred_element_type=jnp.float32)
          + jnp.dot(ph, wh_ref[...], preferred_element_type=jnp.float32)
          + b_ref[0:1, :])
    zr = jax.nn.sigmoid(sv)
    z_ref[...] = zr[:, :D]
    yrh_ref[...] = dv * (zr[:, D:] * h_ref[...])


def _tc_zr(accx, acch, dinv2, h_i, wx, wh, b2):
    bs = pl.BlockSpec((_BM, D), lambda i: (i, 0))
    bd = pl.BlockSpec((_BM, 1), lambda i: (i, 0))
    bw = pl.BlockSpec((D, 2 * D), lambda i: (0, 0))
    bb = pl.BlockSpec((8, 2 * D), lambda i: (0, 0))
    return pl.pallas_call(
        _zr_body,
        grid=(NP // _BM,),
        in_specs=[bs, bs, bd, bs, bw, bw, bb],
        out_specs=[bs, bs],
        out_shape=[jax.ShapeDtypeStruct((NP, D), jnp.float32)] * 2,
    )(accx, acch, dinv2, h_i, wx, wh, b2)


def _ht_body(accx_ref, accrh_ref, dinv_ref, z_ref, h_ref, wx_ref, wh_ref,
             b_ref, hn_ref, y_ref):
    dv = dinv_ref[...]
    px = accx_ref[...] * dv
    prh = accrh_ref[...] * dv
    sv = (jnp.dot(px, wx_ref[...], preferred_element_type=jnp.float32)
          + jnp.dot(prh, wh_ref[...], preferred_element_type=jnp.float32)
          + b_ref[0:1, :])
    ht = jnp.tanh(sv)
    z = z_ref[...]
    hn = z * h_ref[...] + (1.0 - z) * ht
    hn_ref[...] = hn
    y_ref[...] = dv * hn


def _tc_ht(accx, accrh, dinv2, z, h_i, wx, wh, b2):
    bs = pl.BlockSpec((_BM, D), lambda i: (i, 0))
    bd = pl.BlockSpec((_BM, 1), lambda i: (i, 0))
    bw = pl.BlockSpec((D, D), lambda i: (0, 0))
    bb = pl.BlockSpec((8, D), lambda i: (0, 0))
    return pl.pallas_call(
        _ht_body,
        grid=(NP // _BM,),
        in_specs=[bs, bs, bd, bs, bs, bw, bw, bb],
        out_specs=[bs, bs],
        out_shape=[jax.ShapeDtypeStruct((NP, D), jnp.float32)] * 2,
    )(accx, accrh, dinv2, z, h_i, wx, wh, b2)


def kernel(inp, edgidx, h, Wxz, bxz, Whz, bhz, Wxr, bxr, Whr, bhr,
           Wxh, bxh, Whh, bhh):
    src = edgidx[0].astype(jnp.int32)
    dst = edgidx[1].astype(jnp.int32)
    pad = E_PAD - E
    src2 = jnp.concatenate([src, jnp.zeros((pad,), jnp.int32)]).reshape(E_PAD, 1)
    dst2 = jnp.concatenate([dst, jnp.full((pad,), PAD_DST, jnp.int32)]
                           ).reshape(E_PAD, 1)

    pos2, packed2, totals = _tc_rank(src2, dst2)
    counts = totals[0, :NBUK].astype(jnp.int32)
    counts_pad = ((counts + 127) // 128) * 128
    cnts_rep = jnp.repeat(counts, 16)
    cnts_pad_rep = jnp.repeat(counts_pad, 16)

    recs = _sc_scatter(packed2.reshape(-1), pos2.reshape(-1), cnts_rep)
    dinv_rep = _sc_degdinv(recs, cnts_pad_rep)
    dinv2 = dinv_rep[:, :1]

    rowpad = ((0, NP - N), (0, 0))
    yx, yh0, yh1 = _tc_scale3(jnp.pad(inp, rowpad), jnp.pad(h[0], rowpad),
                              jnp.pad(h[1], rowpad), dinv2)

    h_new = []
    yhs = (yh0, yh1)
    y_in = yx
    for i in range(2):
        wx_zr = jnp.concatenate([Wxz[i], Wxr[i]], axis=1)
        wh_zr = jnp.concatenate([Whz[i], Whr[i]], axis=1)
        b_zr = jnp.concatenate([bxz[i] + bhz[i], bxr[i] + bhr[i]])
        b_zr2 = jnp.broadcast_to(b_zr[None, :], (8, 2 * D))
        b_h2 = jnp.broadcast_to((bxh[i] + bhh[i])[None, :], (8, D))
        h_i = jnp.pad(h[i], rowpad)

        accx = _sc_prop(y_in, recs, cnts_pad_rep)
        acch = _sc_prop(yhs[i], recs, cnts_pad_rep)
        z, yrh = _tc_zr(accx, acch, dinv2, h_i, wx_zr, wh_zr, b_zr2)
        accrh = _sc_prop(yrh, recs, cnts_pad_rep)
        hn, y_in = _tc_ht(accx, accrh, dinv2, z, h_i, Wxh[i], Whh[i], b_h2)
        h_new.append(hn[:N])

    h_out = jnp.stack(h_new, axis=0)
    return (h_out, h_out)


# speedup vs baseline: 1.1190x; 1.0044x over previous
import functools

import jax
import jax.numpy as jnp
from jax import lax
from jax.experimental import pallas as pl
from jax.experimental.pallas import tpu as pltpu
from jax.experimental.pallas import tpu_sc as plsc

N = 10000
NP = 10240
D = 256
E = 160000
E_PAD = 163840
PAD_DST = 16384
BROWS = 320
NBUK = 32
CAP = 6144
DUMP = 4096
TOT = NBUK * CAP + DUMP
RB = 512
SUB = 8
G = 32
MAGIC = 0x5F3759DF

_MESH = plsc.VectorSubcoreMesh(core_axis_name="c", subcore_axis_name="s")


def _rank_body(src_ref, dst_ref, pos_ref, packed_ref, tot_ref, carry_ref):
    step = pl.program_id(0)

    @pl.when(step == 0)
    def _():
        carry_ref[...] = jnp.zeros((1, 128), jnp.float32)

    col = lax.broadcasted_iota(jnp.int32, (RB, 128), 1)
    rr = lax.broadcasted_iota(jnp.int32, (RB, RB), 0)
    cc = lax.broadcasted_iota(jnp.int32, (RB, RB), 1)
    ltri = (rr > cc).astype(jnp.float32)
    ones = jnp.ones((128, 128), jnp.float32)
    for sb in range(SUB):
        d = dst_ref[pl.ds(sb * RB, RB), :]
        s = src_ref[pl.ds(sb * RB, RB), :]
        b = jnp.minimum((d * 6554) >> 21, NBUK)
        l = d - b * BROWS
        ind = (b == col).astype(jnp.float32)
        rank = jnp.dot(ltri, ind, preferred_element_type=jnp.float32)
        carry = carry_ref[...]
        rel = jnp.sum(ind * (rank + carry), axis=1, keepdims=True)
        carry_ref[...] = carry + jnp.sum(ind, axis=0, keepdims=True)
        pos_ref[pl.ds(sb * RB, RB), :] = rel.astype(jnp.int32) + b * CAP
        packed_ref[pl.ds(sb * RB, RB), :] = s * 512 + l
    tot_ref[...] = carry_ref[...]


def _tc_rank(src2, dst2):
    nstep = E_PAD // (RB * SUB)
    be = pl.BlockSpec((RB * SUB, 1), lambda i: (i, 0))
    bt = pl.BlockSpec((1, 128), lambda i: (0, 0))
    return pl.pallas_call(
        _rank_body,
        grid=(nstep,),
        in_specs=[be, be],
        out_specs=[be, be, bt],
        out_shape=[
            jax.ShapeDtypeStruct((E_PAD, 1), jnp.int32),
            jax.ShapeDtypeStruct((E_PAD, 1), jnp.int32),
            jax.ShapeDtypeStruct((1, 128), jnp.float32),
        ],
        scratch_shapes=[pltpu.VMEM((1, 128), jnp.float32)],
    )(src2, dst2)


@functools.partial(
    pl.kernel,
    out_type=jax.ShapeDtypeStruct((TOT,), jnp.int32),
    mesh=_MESH,
    scratch_types=[
        pltpu.VMEM((E_PAD // 32 // 128, 128), jnp.int32),
        pltpu.VMEM((E_PAD // 32 // 128, 128), jnp.int32),
        pltpu.VMEM((64,), jnp.int32),
        pltpu.VMEM((64,), jnp.int32),
        pltpu.VMEM((16,), jnp.int32),
        pltpu.SemaphoreType.DMA,
    ],
)
def _sc_scatter(packed_hbm, pos_hbm, cnts_hbm, recs_hbm,
                pos_v, val_v, fidx_v, fval_v, cnt_v, sem0):
    c = lax.axis_index("c")
    s = lax.axis_index("s")
    wid = c * 16 + s
    nchunk = E_PAD // 32 // 128
    pltpu.sync_copy(pos_hbm.at[pl.ds(wid * nchunk, nchunk)], pos_v)
    pltpu.sync_copy(packed_hbm.at[pl.ds(wid * nchunk, nchunk)], val_v)

    def fire(i, carry):
        pltpu.async_copy(val_v.at[i], recs_hbm.at[pos_v.at[i]], sem0)
        return carry

    lax.fori_loop(0, nchunk, fire, 0)

    def drain(i, carry):
        pltpu.make_async_copy(val_v.at[0], recs_hbm.at[pos_v.at[0]],
                              sem0).wait()
        return carry

    lax.fori_loop(0, nchunk, drain, 0)

    pltpu.sync_copy(cnts_hbm.at[pl.ds(wid * 16, 16)], cnt_v)
    cnt = cnt_v[...][0]
    iota16 = lax.iota(jnp.int32, 16)
    dummy = jnp.full((16,), BROWS, jnp.int32)
    for half in range(2):
        for t in range(4):
            fidx_v[pl.ds(t * 16, 16)] = (wid * CAP + cnt + (half * 4 + t) * 16
                                         + iota16)
            fval_v[pl.ds(t * 16, 16)] = dummy
        pltpu.sync_copy(fval_v, recs_hbm.at[fidx_v])


@functools.partial(
    pl.kernel,
    out_type=jax.ShapeDtypeStruct((NP, 16), jnp.float32),
    mesh=_MESH,
    scratch_types=[
        pltpu.VMEM((CAP,), jnp.int32),
        pltpu.VMEM((16,), jnp.int32),
        pltpu.VMEM((336, 16), jnp.float32),
        pltpu.VMEM((BROWS, 16), jnp.float32),
    ],
)
def _sc_degdinv(recs_hbm, cnts_hbm, dinv_hbm, rec_v, cnt_v, deg_v, dv_v):
    c = lax.axis_index("c")
    s = lax.axis_index("s")
    wid = c * 16 + s
    pltpu.sync_copy(recs_hbm.at[pl.ds(wid * CAP, CAP)], rec_v)
    pltpu.sync_copy(cnts_hbm.at[pl.ds(wid * 16, 16)], cnt_v)
    cntp = cnt_v[...][0]
    one16 = jnp.ones((16,), jnp.float32)

    def init_body(k, carry):
        for t in range(4):
            deg_v[k * 4 + t, :] = one16
        return carry

    lax.fori_loop(0, 84, init_body, 0)

    def edge_body(j, carry):
        p16 = rec_v[pl.ds(j * 16, 16)]
        l16 = jnp.minimum(p16 & 511, BROWS)
        for e in range(16):
            le = l16[e]
            deg_v[le, :] = deg_v[le, :] + 1.0
        return carry

    lax.fori_loop(0, cntp // 16, edge_body, 0)

    def dinv_body(k, carry):
        for t in range(4):
            r = k * 4 + t
            dd = deg_v[r, :]
            ib = lax.bitcast_convert_type(dd, jnp.int32)
            ib = jnp.int32(MAGIC) - (ib >> 1)
            x = lax.bitcast_convert_type(ib, jnp.float32)
            for _ in range(3):
                x = x * (1.5 - 0.5 * dd * x * x)
            dv_v[r, :] = x
        return carry

    lax.fori_loop(0, BROWS // 4, dinv_body, 0)
    pltpu.sync_copy(dv_v, dinv_hbm.at[pl.ds(wid * BROWS, BROWS)])


@functools.partial(
    pl.kernel,
    out_type=jax.ShapeDtypeStruct((NP, D), jnp.float32),
    mesh=_MESH,
    scratch_types=[
        pltpu.VMEM((CAP,), jnp.int32),
        pltpu.VMEM((16,), jnp.int32),
        pltpu.VMEM((2 * G,), jnp.int32),
        pltpu.VMEM((G, D), jnp.float32),
        pltpu.VMEM((G, D), jnp.float32),
        pltpu.VMEM((328, D), jnp.float32),
        pltpu.SemaphoreType.DMA,
        pltpu.SemaphoreType.DMA,
    ],
)
def _sc_prop(y_hbm, recs_hbm, cnts_hbm, out_hbm,
             rec_v, cnt_v, gidx_v, stage0, stage1, acc_v, sem0, sem1):
    c = lax.axis_index("c")
    s = lax.axis_index("s")
    wid = c * 16 + s
    base_row = wid * BROWS
    pltpu.sync_copy(recs_hbm.at[pl.ds(wid * CAP, CAP)], rec_v)
    pltpu.sync_copy(cnts_hbm.at[pl.ds(wid * 16, 16)], cnt_v)
    cntp = cnt_v[...][0]
    ngrp = cntp // G
    gmax = jnp.maximum(ngrp - 1, 0)
    pltpu.sync_copy(y_hbm.at[pl.ds(base_row, BROWS)], acc_v.at[pl.ds(0, BROWS)])

    stages = (stage0, stage1)
    sems = (sem0, sem1)

    def fire(g, slot):
        gg = jnp.minimum(g, gmax) * G
        for t in range(G // 16):
            p16 = rec_v[pl.ds(gg + t * 16, 16)]
            gidx_v[pl.ds(slot * G + t * 16, 16)] = jnp.minimum(p16 >> 9, NP - 1)
        pltpu.async_copy(y_hbm.at[gidx_v.at[pl.ds(slot * G, G)]],
                         stages[slot], sems[slot])

    def wait(slot):
        pltpu.make_async_copy(y_hbm.at[gidx_v.at[pl.ds(slot * G, G)]],
                              stages[slot], sems[slot]).wait()

    def process(g, slot):
        st = stages[slot]
        les = []
        for t in range(G // 16):
            p16 = rec_v[pl.ds(g * G + t * 16, 16)]
            l16 = jnp.minimum(p16 & 511, BROWS)
            for e in range(16):
                les.append(l16[e])
        for e in range(G):
            le = les[e]
            vals = [acc_v[le, pl.ds(k * 16, 16)] for k in range(16)]
            svals = [st[e, pl.ds(k * 16, 16)] for k in range(16)]
            for k in range(16):
                acc_v[le, pl.ds(k * 16, 16)] = vals[k] + svals[k]

    @pl.when(ngrp > 0)
    def _():
        fire(0, 0)
        fire(1, 1)

        def body(j, carry):
            for b in range(2):
                g = j * 2 + b
                wait(b)
                process(g, b)
                fire(g + 2, b)
            return carry

        lax.fori_loop(0, ngrp // 2, body, 0)
        wait(0)
        wait(1)

    pltpu.sync_copy(acc_v.at[pl.ds(0, BROWS)],
                    out_hbm.at[pl.ds(base_row, BROWS)])


_BM = 1024


def _scale3_body(inp_ref, h0_ref, h1_ref, dinv_ref, yx_ref, yh0_ref, yh1_ref):
    dv = dinv_ref[...]
    yx_ref[...] = inp_ref[...] * dv
    yh0_ref[...] = h0_ref[...] * dv
    yh1_ref[...] = h1_ref[...] * dv


def _tc_scale3(inp, h0, h1, dinv2):
    bs = pl.BlockSpec((_BM, D), lambda i: (i, 0))
    bd = pl.BlockSpec((_BM, 1), lambda i: (i, 0))
    return pl.pallas_call(
        _scale3_body,
        grid=(NP // _BM,),
        in_specs=[bs, bs, bs, bd],
        out_specs=[bs, bs, bs],
        out_shape=[jax.ShapeDtypeStruct((NP, D), jnp.float32)] * 3,
    )(inp, h0, h1, dinv2)


def _zr_body(accx_ref, acch_ref, dinv_ref, h_ref, wx_ref, wh_ref, b_ref,
             z_ref, yrh_ref):
    dv = dinv_ref[...]
    px = accx_ref[...] * dv
    ph = acch_ref[...] * dv
    sv = (jnp.dot(px, wx_ref[...], preferred_element_type=jnp.float32)
          + jnp.dot(ph, wh_ref[...], preferred_element_type=jnp.float32)
          + b_ref[0:1, :])
    zr = jax.nn.sigmoid(sv)
    z_ref[...] = zr[:, :D]
    yrh_ref[...] = dv * (zr[:, D:] * h_ref[...])


def _tc_zr(accx, acch, dinv2, h_i, wx, wh, b2):
    bs = pl.BlockSpec((_BM, D), lambda i: (i, 0))
    bd = pl.BlockSpec((_BM, 1), lambda i: (i, 0))
    bw = pl.BlockSpec((D, 2 * D), lambda i: (0, 0))
    bb = pl.BlockSpec((8, 2 * D), lambda i: (0, 0))
    return pl.pallas_call(
        _zr_body,
        grid=(NP // _BM,),
        in_specs=[bs, bs, bd, bs, bw, bw, bb],
        out_specs=[bs, bs],
        out_shape=[jax.ShapeDtypeStruct((NP, D), jnp.float32)] * 2,
    )(accx, acch, dinv2, h_i, wx, wh, b2)


def _ht_body(accx_ref, accrh_ref, dinv_ref, z_ref, h_ref, wx_ref, wh_ref,
             b_ref, hn_ref, y_ref):
    dv = dinv_ref[...]
    px = accx_ref[...] * dv
    prh = accrh_ref[...] * dv
    sv = (jnp.dot(px, wx_ref[...], preferred_element_type=jnp.float32)
          + jnp.dot(prh, wh_ref[...], preferred_element_type=jnp.float32)
          + b_ref[0:1, :])
    ht = jnp.tanh(sv)
    z = z_ref[...]
    hn = z * h_ref[...] + (1.0 - z) * ht
    hn_ref[...] = hn
    y_ref[...] = dv * hn


def _tc_ht(accx, accrh, dinv2, z, h_i, wx, wh, b2):
    bs = pl.BlockSpec((_BM, D), lambda i: (i, 0))
    bd = pl.BlockSpec((_BM, 1), lambda i: (i, 0))
    bw = pl.BlockSpec((D, D), lambda i: (0, 0))
    bb = pl.BlockSpec((8, D), lambda i: (0, 0))
    return pl.pallas_call(
        _ht_body,
        grid=(NP // _BM,),
        in_specs=[bs, bs, bd, bs, bs, bw, bw, bb],
        out_specs=[bs, bs],
        out_shape=[jax.ShapeDtypeStruct((NP, D), jnp.float32)] * 2,
    )(accx, accrh, dinv2, z, h_i, wx, wh, b2)


def kernel(inp, edgidx, h, Wxz, bxz, Whz, bhz, Wxr, bxr, Whr, bhr,
           Wxh, bxh, Whh, bhh):
    src = edgidx[0].astype(jnp.int32)
    dst = edgidx[1].astype(jnp.int32)
    pad = E_PAD - E
    src2 = jnp.concatenate([src, jnp.zeros((pad,), jnp.int32)]).reshape(E_PAD, 1)
    dst2 = jnp.concatenate([dst, jnp.full((pad,), PAD_DST, jnp.int32)]
                           ).reshape(E_PAD, 1)

    pos2, packed2, totals = _tc_rank(src2, dst2)
    counts = totals[0, :NBUK].astype(jnp.int32)
    counts_pad = ((counts + 127) // 128) * 128
    cnts_rep = jnp.repeat(counts, 16)
    cnts_pad_rep = jnp.repeat(counts_pad, 16)

    recs = _sc_scatter(packed2.reshape(-1, 128), pos2.reshape(-1, 128),
                       cnts_rep)
    dinv_rep = _sc_degdinv(recs, cnts_pad_rep)
    dinv2 = dinv_rep[:, :1]

    rowpad = ((0, NP - N), (0, 0))
    yx, yh0, yh1 = _tc_scale3(jnp.pad(inp, rowpad), jnp.pad(h[0], rowpad),
                              jnp.pad(h[1], rowpad), dinv2)

    h_new = []
    yhs = (yh0, yh1)
    y_in = yx
    for i in range(2):
        wx_zr = jnp.concatenate([Wxz[i], Wxr[i]], axis=1)
        wh_zr = jnp.concatenate([Whz[i], Whr[i]], axis=1)
        b_zr = jnp.concatenate([bxz[i] + bhz[i], bxr[i] + bhr[i]])
        b_zr2 = jnp.broadcast_to(b_zr[None, :], (8, 2 * D))
        b_h2 = jnp.broadcast_to((bxh[i] + bhh[i])[None, :], (8, D))
        h_i = jnp.pad(h[i], rowpad)

        accx = _sc_prop(y_in, recs, cnts_pad_rep)
        acch = _sc_prop(yhs[i], recs, cnts_pad_rep)
        z, yrh = _tc_zr(accx, acch, dinv2, h_i, wx_zr, wh_zr, b_zr2)
        accrh = _sc_prop(yrh, recs, cnts_pad_rep)
        hn, y_in = _tc_ht(accx, accrh, dinv2, z, h_i, Wxh[i], Whh[i], b_h2)
        h_new.append(hn[:N])

    h_out = jnp.stack(h_new, axis=0)
    return (h_out, h_out)
